# baseline (speedup 1.0000x reference)
import jax
import jax.numpy as jnp
from jax import lax
from jax.experimental import pallas as pl
from jax.experimental.pallas import tpu as pltpu
from jax.experimental.pallas import tpu_sc as plsc

N = 10000
E = 320000
F_IN = 128
F_PRIME = 256
HALF = 128
NTILES = 16
EDGES_TOTAL = 2 * E
EDGES_PER_TILE = EDGES_TOTAL // NTILES
K = 80
NBUF = 4
CPS = 20
SUP = K * CPS
NSUP = EDGES_PER_TILE // SUP
NPAD = 10240
ROWS_PER_TILE = NPAD // NTILES
DUMMY_ROW0 = N


def _matmul_body(x_ref, w_ref, o_ref):
    o_ref[0] = jnp.dot(x_ref[...], w_ref[0], preferred_element_type=jnp.float32)


def _hidden_tc(xf, wstack):
    nblk = 10
    bn = N // nblk
    out = pl.pallas_call(
        _matmul_body,
        grid=(4, nblk),
        in_specs=[
            pl.BlockSpec((bn, F_IN), lambda i, j: (j, 0)),
            pl.BlockSpec((1, F_IN, HALF), lambda i, j: (i // 2, 0, i % 2)),
        ],
        out_specs=pl.BlockSpec((1, bn, HALF), lambda i, j: (i, j, 0)),
        out_shape=jax.ShapeDtypeStruct((4, N, HALF), jnp.float32),
    )(xf, wstack)
    return out.reshape(4 * N, HALF)


def _bcast_lane(vv, l):
    return lax.gather(
        vv, jnp.full((16, 1), l, jnp.int32),
        dimension_numbers=lax.GatherDimensionNumbers(
            offset_dims=(), collapsed_slice_dims=(0,),
            start_index_map=(0,)),
        slice_sizes=(1,),
        mode=lax.GatherScatterMode.PROMISE_IN_BOUNDS)


def _sc_body(hid_ref, ei0_ref, ev0_ref, ei1_ref, ev1_ref, out_ref,
             acc, cbig, rbig, vbig,
             cb0, cb1, cb2, cb3, rb0, rb1, rb2, rb3,
             gb0, gb1, gb2, gb3,
             gs0, gs1, gs2, gs3, ws0, ws1, ws2, ws3):
    cbufs = [cb0, cb1, cb2, cb3]
    rbufs = [rb0, rb1, rb2, rb3]
    gbufs = [gb0, gb1, gb2, gb3]
    gsems = [gs0, gs1, gs2, gs3]
    wsems = [ws0, ws1, ws2, ws3]

    c = lax.axis_index("c")
    s = lax.axis_index("s")
    col_off = (c * N).astype(jnp.int32)

    zero16 = jnp.zeros((16,), jnp.float32)

    def zrow(i, _):
        for b in range(NBUF):
            for t in range(HALF // 16):
                gbufs[b][i, pl.ds(t * 16, 16)] = zero16
        return 0
    lax.fori_loop(0, K, zrow, 0)
    row0 = s * ROWS_PER_TILE
    for kblk in range(ROWS_PER_TILE // K):
        pltpu.sync_copy(gbufs[kblk % NBUF],
                        acc.at[pl.ds(row0 + kblk * K, K)])
    plsc.subcore_barrier()

    for b in range(NBUF):
        for t in range(K // 16):
            rbufs[b][pl.ds(t * 16, 16)] = (
                lax.iota(jnp.int32, 16) + (DUMMY_ROW0 + t * 16))
        pltpu.async_copy(gbufs[b], acc.at[rbufs[b]], wsems[b], add=True)

    def wait_scatter(b):
        pltpu.make_async_copy(gbufs[b], acc.at[rbufs[b]], wsems[b]).wait()

    def prep_and_gather(b, off, foff):
        for t in range(K // 16):
            cbufs[b][pl.ds(t * 16, 16)] = (
                cbig[pl.ds(off + t * 16, 16)] + (col_off + foff))
            rbufs[b][pl.ds(t * 16, 16)] = rbig[pl.ds(off + t * 16, 16)]
        pltpu.async_copy(hid_ref.at[cbufs[b]], gbufs[b], gsems[b])

    def wait_gather(b):
        pltpu.make_async_copy(hid_ref.at[cbufs[b]], gbufs[b], gsems[b]).wait()

    def scale(b, ch):
        for gg in range(K // 16):
            vv = vbig[pl.ds(ch * K + gg * 16, 16)]
            for l in range(16):
                evb = _bcast_lane(vv, l)
                i = gg * 16 + l
                for t in range(HALF // 16):
                    g = gbufs[b][i, pl.ds(t * 16, 16)]
                    gbufs[b][i, pl.ds(t * 16, 16)] = g * evb

    def process(ei_ref, ev_ref, tile_idx, foff):
        def superblock(sb, _):
            ebase = tile_idx * EDGES_PER_TILE + sb * SUP
            pltpu.sync_copy(ei_ref.at[pl.ds(E + ebase, SUP)], cbig)
            pltpu.sync_copy(ei_ref.at[pl.ds(ebase, SUP)], rbig)
            pltpu.sync_copy(ev_ref.at[pl.ds(ebase, SUP)], vbig)

            for b in range(NBUF - 1):
                wait_scatter(b)
                prep_and_gather(b, b * K, foff)

            def block(blk, _):
                c0 = blk * NBUF
                for b in range(NBUF):
                    ch = c0 + b
                    wait_gather(b)
                    scale(b, ch)
                    pltpu.async_copy(gbufs[b], acc.at[rbufs[b]], wsems[b],
                                     add=True)
                    pb = (b + NBUF - 1) % NBUF
                    nxt = ch + NBUF - 1

                    @pl.when(nxt < CPS)
                    def _():
                        wait_scatter(pb)
                        prep_and_gather(pb, nxt * K, foff)
                return 0
            lax.fori_loop(0, CPS // NBUF, block, 0)
            return 0
        return superblock

    nB = (s // 8) * NSUP
    lax.fori_loop(0, NSUP - nB, process(ei0_ref, ev0_ref, s, 0), 0)
    lax.fori_loop(0, nB, process(ei1_ref, ev1_ref, s - 8, 2 * N), 0)

    for b in range(NBUF):
        wait_scatter(b)
    plsc.subcore_barrier()

    pltpu.sync_copy(acc.at[pl.ds(row0, ROWS_PER_TILE)],
                    out_ref.at[pl.ds(row0, ROWS_PER_TILE),
                               pl.ds(c * HALF, HALF)])


@jax.jit
def kernel(x, edge_index0, edge_vals0, edge_index1, edge_vals1, W0, W1):
    xf = x.reshape(N, F_IN)
    wstack = jnp.stack([W0, W1])
    hid = _hidden_tc(xf, wstack)

    mesh = plsc.VectorSubcoreMesh(core_axis_name="c", subcore_axis_name="s")
    sc = pl.kernel(
        _sc_body,
        out_type=jax.ShapeDtypeStruct((NPAD, F_PRIME), jnp.float32),
        mesh=mesh,
        scratch_types=[
            pltpu.VMEM_SHARED((NPAD, HALF), jnp.float32),
            pltpu.VMEM((SUP,), jnp.int32),
            pltpu.VMEM((SUP,), jnp.int32),
            pltpu.VMEM((SUP,), jnp.float32),
            pltpu.VMEM((K,), jnp.int32),
            pltpu.VMEM((K,), jnp.int32),
            pltpu.VMEM((K,), jnp.int32),
            pltpu.VMEM((K,), jnp.int32),
            pltpu.VMEM((K,), jnp.int32),
            pltpu.VMEM((K,), jnp.int32),
            pltpu.VMEM((K,), jnp.int32),
            pltpu.VMEM((K,), jnp.int32),
            pltpu.VMEM((K, HALF), jnp.float32),
            pltpu.VMEM((K, HALF), jnp.float32),
            pltpu.VMEM((K, HALF), jnp.float32),
            pltpu.VMEM((K, HALF), jnp.float32),
            pltpu.SemaphoreType.DMA,
            pltpu.SemaphoreType.DMA,
            pltpu.SemaphoreType.DMA,
            pltpu.SemaphoreType.DMA,
            pltpu.SemaphoreType.DMA,
            pltpu.SemaphoreType.DMA,
            pltpu.SemaphoreType.DMA,
            pltpu.SemaphoreType.DMA,
        ],
    )
    out = sc(hid, edge_index0.reshape(-1), edge_vals0,
             edge_index1.reshape(-1), edge_vals1)
    return out[:N].reshape(1, N, F_PRIME)


# speedup vs baseline: 2.0045x; 2.0045x over previous
import jax
import jax.numpy as jnp
from jax import lax
from jax.experimental import pallas as pl
from jax.experimental.pallas import tpu as pltpu
from jax.experimental.pallas import tpu_sc as plsc

N = 10000
E = 320000
F_IN = 128
F_PRIME = 256
HALF = 128
NTILES = 16
EDGES_TOTAL = 2 * E
EDGES_PER_TILE = EDGES_TOTAL // NTILES
K = 80
NBUF = 4
CPS = 20
SUP = K * CPS
NSUP = EDGES_PER_TILE // SUP
NPAD = 10240
ROWS_PER_TILE = NPAD // NTILES
DUMMY_ROW0 = N


def _matmul_body(x_ref, w_ref, o_ref):
    o_ref[0] = jnp.dot(x_ref[...], w_ref[0], preferred_element_type=jnp.float32)


def _hidden_tc(xf, wstack):
    nblk = 10
    bn = N // nblk
    out = pl.pallas_call(
        _matmul_body,
        grid=(4, nblk),
        in_specs=[
            pl.BlockSpec((bn, F_IN), lambda i, j: (j, 0)),
            pl.BlockSpec((1, F_IN, HALF), lambda i, j: (i // 2, 0, i % 2)),
        ],
        out_specs=pl.BlockSpec((1, bn, HALF), lambda i, j: (i, j, 0)),
        out_shape=jax.ShapeDtypeStruct((4, N, HALF), jnp.float32),
    )(xf, wstack)
    return out.reshape(4 * N, HALF)


def _bcast_lane(vv, l):
    return lax.gather(
        vv, jnp.full((16, 1), l, jnp.int32),
        dimension_numbers=lax.GatherDimensionNumbers(
            offset_dims=(), collapsed_slice_dims=(0,),
            start_index_map=(0,)),
        slice_sizes=(1,),
        mode=lax.GatherScatterMode.PROMISE_IN_BOUNDS)


def _sc_body(hid_ref, ei0_ref, ev0_ref, ei1_ref, ev1_ref, out_ref,
             acc, cbig, rbig, vbig,
             cb0, cb1, cb2, cb3, rb0, rb1, rb2, rb3,
             gb0, gb1, gb2, gb3,
             gs0, gs1, gs2, gs3, ws0, ws1, ws2, ws3):
    cbufs = [cb0, cb1, cb2, cb3]
    rbufs = [rb0, rb1, rb2, rb3]
    gbufs = [gb0, gb1, gb2, gb3]
    gsems = [gs0, gs1, gs2, gs3]
    wsems = [ws0, ws1, ws2, ws3]

    c = lax.axis_index("c")
    s = lax.axis_index("s")
    col_off = (c * N).astype(jnp.int32)

    zero16 = jnp.zeros((16,), jnp.float32)

    def zrow(i, _):
        for b in range(NBUF):
            for t in range(HALF // 16):
                gbufs[b][i, pl.ds(t * 16, 16)] = zero16
        return 0
    lax.fori_loop(0, K, zrow, 0)
    row0 = s * ROWS_PER_TILE
    for kblk in range(ROWS_PER_TILE // K):
        pltpu.sync_copy(gbufs[kblk % NBUF],
                        acc.at[pl.ds(row0 + kblk * K, K)])
    plsc.subcore_barrier()

    for b in range(NBUF):
        for t in range(K // 16):
            rbufs[b][pl.ds(t * 16, 16)] = (
                lax.iota(jnp.int32, 16) + (DUMMY_ROW0 + t * 16))
        pltpu.async_copy(gbufs[b], acc.at[rbufs[b]], wsems[b], add=True)

    def wait_scatter(b):
        pltpu.make_async_copy(gbufs[b], acc.at[rbufs[b]], wsems[b]).wait()

    def prep_and_gather(b, off, foff):
        for t in range(K // 16):
            cbufs[b][pl.ds(t * 16, 16)] = (
                cbig[pl.ds(off + t * 16, 16)] + (col_off + foff))
            rbufs[b][pl.ds(t * 16, 16)] = rbig[pl.ds(off + t * 16, 16)]
        pltpu.async_copy(hid_ref.at[cbufs[b]], gbufs[b], gsems[b])

    def wait_gather(b):
        pltpu.make_async_copy(hid_ref.at[cbufs[b]], gbufs[b], gsems[b]).wait()

    def scale(b, ch):
        def group(gg, _):
            vv = vbig[pl.ds(ch * K + gg * 16, 16)]
            base = gg * 16
            for l in range(16):
                evb = _bcast_lane(vv, l)
                i = base + l
                for t in range(HALF // 16):
                    g = gbufs[b][i, pl.ds(t * 16, 16)]
                    gbufs[b][i, pl.ds(t * 16, 16)] = g * evb
            return 0
        lax.fori_loop(0, K // 16, group, 0)

    def process(ei_ref, ev_ref, tile_idx, foff):
        def superblock(sb, _):
            ebase = tile_idx * EDGES_PER_TILE + sb * SUP
            pltpu.sync_copy(ei_ref.at[pl.ds(E + ebase, SUP)], cbig)
            pltpu.sync_copy(ei_ref.at[pl.ds(ebase, SUP)], rbig)
            pltpu.sync_copy(ev_ref.at[pl.ds(ebase, SUP)], vbig)

            for b in range(NBUF - 1):
                wait_scatter(b)
                prep_and_gather(b, b * K, foff)

            def block(blk, _):
                c0 = blk * NBUF
                for b in range(NBUF):
                    ch = c0 + b
                    wait_gather(b)
                    scale(b, ch)
                    pltpu.async_copy(gbufs[b], acc.at[rbufs[b]], wsems[b],
                                     add=True)
                    pb = (b + NBUF - 1) % NBUF
                    nxt = ch + NBUF - 1

                    @pl.when(nxt < CPS)
                    def _():
                        wait_scatter(pb)
                        prep_and_gather(pb, nxt * K, foff)
                return 0
            lax.fori_loop(0, CPS // NBUF, block, 0)
            return 0
        return superblock

    nB = (s // 8) * NSUP
    lax.fori_loop(0, NSUP - nB, process(ei0_ref, ev0_ref, s, 0), 0)
    lax.fori_loop(0, nB, process(ei1_ref, ev1_ref, s - 8, 2 * N), 0)

    for b in range(NBUF):
        wait_scatter(b)
    plsc.subcore_barrier()

    pltpu.sync_copy(acc.at[pl.ds(row0, ROWS_PER_TILE)],
                    out_ref.at[pl.ds(row0, ROWS_PER_TILE),
                               pl.ds(c * HALF, HALF)])


@jax.jit
def kernel(x, edge_index0, edge_vals0, edge_index1, edge_vals1, W0, W1):
    xf = x.reshape(N, F_IN)
    wstack = jnp.stack([W0, W1])
    hid = _hidden_tc(xf, wstack)

    mesh = plsc.VectorSubcoreMesh(core_axis_name="c", subcore_axis_name="s")
    sc = pl.kernel(
        _sc_body,
        out_type=jax.ShapeDtypeStruct((NPAD, F_PRIME), jnp.float32),
        mesh=mesh,
        scratch_types=[
            pltpu.VMEM_SHARED((NPAD, HALF), jnp.float32),
            pltpu.VMEM((SUP,), jnp.int32),
            pltpu.VMEM((SUP,), jnp.int32),
            pltpu.VMEM((SUP,), jnp.float32),
            pltpu.VMEM((K,), jnp.int32),
            pltpu.VMEM((K,), jnp.int32),
            pltpu.VMEM((K,), jnp.int32),
            pltpu.VMEM((K,), jnp.int32),
            pltpu.VMEM((K,), jnp.int32),
            pltpu.VMEM((K,), jnp.int32),
            pltpu.VMEM((K,), jnp.int32),
            pltpu.VMEM((K,), jnp.int32),
            pltpu.VMEM((K, HALF), jnp.float32),
            pltpu.VMEM((K, HALF), jnp.float32),
            pltpu.VMEM((K, HALF), jnp.float32),
            pltpu.VMEM((K, HALF), jnp.float32),
            pltpu.SemaphoreType.DMA,
            pltpu.SemaphoreType.DMA,
            pltpu.SemaphoreType.DMA,
            pltpu.SemaphoreType.DMA,
            pltpu.SemaphoreType.DMA,
            pltpu.SemaphoreType.DMA,
            pltpu.SemaphoreType.DMA,
            pltpu.SemaphoreType.DMA,
        ],
    )
    out = sc(hid, edge_index0.reshape(-1), edge_vals0,
             edge_index1.reshape(-1), edge_vals1)
    return out[:N].reshape(1, N, F_PRIME)
